# Initial kernel scaffold; baseline (speedup 1.0000x reference)
import jax
import jax.numpy as jnp
from jax.experimental import pallas as pl


def kernel(geom, aux, batch, W1, b1, bn1_g, bn1_b, W2, b2, bn2_g, bn2_b, Wa1, ba1, Wa2, ba2, ln_g, ln_b):
    raise NotImplementedError("write your pallas kernel here")


# speedup vs baseline: 4.8065x; 4.8065x over previous
import functools

import jax
import jax.numpy as jnp
from jax import lax
from jax.experimental import pallas as pl
from jax.experimental.pallas import tpu as pltpu
from jax.experimental.pallas import tpu_sc as plsc

_N = 10000
_NP = 10240
_K = 20
_C = 512
_NCHUNK = _NP // _C
_RB = 256
_BN = 400
_NBLK = _N // _BN
_E = _N * _K
_NW = 32
_CB = 256
_NCH_SC = (_K * _NP) // (_NW * _CB)


def _knn_body(lo_ref, hi_ref, g_ref, gt_ref, brow_ref, bcol_ref, nbr_ref,
              dist_ref):
    i = pl.program_id(0)
    g = g_ref[...]
    sq_r = jnp.sum(g * g, axis=1, keepdims=True)
    row_ids = i * _RB + lax.broadcasted_iota(jnp.int32, (_RB, 1), 0)
    brow = brow_ref[...]
    lo = lo_ref[i]
    hi = hi_ref[i]
    clo = lo // _C
    chi = (hi + _C - 1) // _C

    def dist_body(c, _):
        gt = gt_ref[c]
        d = -2.0 * jnp.dot(g, gt, preferred_element_type=jnp.float32)
        sq_c = jnp.sum(gt * gt, axis=0, keepdims=True)
        d = d + sq_r + sq_c
        bcol = bcol_ref[c]
        col_ids = c * _C + lax.broadcasted_iota(jnp.int32, (_RB, _C), 1)
        bad = (brow != bcol) | (col_ids == row_ids)
        dist_ref[c] = jnp.where(bad, jnp.inf, d)
        return 0

    lax.fori_loop(clo, chi, dist_body, 0)

    bigi = jnp.int32(2**30)
    m_prev = jnp.full((_RB, 1), -jnp.inf, jnp.float32)
    i_prev = jnp.full((_RB, 1), -1, jnp.int32)
    for k in range(_K):
        def sel_body(c, carry, m_prev=m_prev, i_prev=i_prev):
            bv, bi = carry
            d = dist_ref[c]
            col_ids = c * _C + lax.broadcasted_iota(jnp.int32, (_RB, _C), 1)
            valid = (d > m_prev) | ((d == m_prev) & (col_ids > i_prev))
            dm = jnp.where(valid, d, jnp.inf)
            cmin = jnp.min(dm, axis=1, keepdims=True)
            cidx = jnp.min(jnp.where((dm == cmin) & valid, col_ids, bigi),
                           axis=1, keepdims=True)
            take = (cmin < bv) | ((cmin == bv) & (cidx < bi))
            return jnp.where(take, cmin, bv), jnp.where(take, cidx, bi)

        best_v, best_i = lax.fori_loop(
            clo, chi, sel_body,
            (jnp.full((_RB, 1), jnp.inf, jnp.float32),
             jnp.full((_RB, 1), bigi, jnp.int32)))
        nbr_ref[:, k:k + 1] = best_i
        m_prev, i_prev = best_v, best_i


def _knn(lo, hi, geom_pad, gt, brow, bcol):
    return pl.pallas_call(
        _knn_body,
        grid=(_NP // _RB,),
        in_specs=[
            pl.BlockSpec(memory_space=pltpu.SMEM),
            pl.BlockSpec(memory_space=pltpu.SMEM),
            pl.BlockSpec((_RB, 128), lambda i: (i, 0)),
            pl.BlockSpec((_NCHUNK, 128, _C), lambda i: (0, 0, 0)),
            pl.BlockSpec((_RB, 1), lambda i: (i, 0)),
            pl.BlockSpec((_NCHUNK, 1, _C), lambda i: (0, 0, 0)),
        ],
        out_specs=pl.BlockSpec((_RB, 128), lambda i: (i, 0)),
        out_shape=jax.ShapeDtypeStruct((_NP, 128), jnp.int32),
        scratch_shapes=[pltpu.VMEM((_NCHUNK, _RB, _C), jnp.float32)],
    )(lo, hi, geom_pad, gt, brow, bcol)


def _prep_body(g_ref, w1_ref, b1_ref, p_ref, q_ref):
    wa = w1_ref[0:128, :]
    wb = w1_ref[128:256, :]
    g = g_ref[...]
    p_ref[...] = (jnp.dot(g, wa - wb, preferred_element_type=jnp.float32)
                  + b1_ref[...])
    q_ref[...] = jnp.dot(g, wb, preferred_element_type=jnp.float32)


def _prep(geom, W1, b1r):
    return pl.pallas_call(
        _prep_body,
        out_shape=(jax.ShapeDtypeStruct((_N, 128), jnp.float32),
                   jax.ShapeDtypeStruct((_N, 128), jnp.float32)),
    )(geom, W1, b1r)


def _sc_gather_call(qtab, auxtab, src):
    mesh = plsc.VectorSubcoreMesh(core_axis_name="c", subcore_axis_name="s")

    @functools.partial(
        pl.kernel,
        mesh=mesh,
        out_type=(jax.ShapeDtypeStruct((_NW, _NCH_SC, _CB, 128), jnp.float32),
                  jax.ShapeDtypeStruct((_NW, _NCH_SC, _CB, 128), jnp.float32)),
        scratch_types=[
            pltpu.VMEM((_CB,), jnp.int32),
            pltpu.VMEM((_CB, 128), jnp.float32),
            pltpu.VMEM((_CB, 128), jnp.float32),
            pltpu.SemaphoreType.DMA,
            pltpu.SemaphoreType.DMA,
        ],
    )
    def gather_kernel(qtab_hbm, auxtab_hbm, src_hbm, qout_hbm, aout_hbm,
                      idx_v, qrow_v, arow_v, sem1, sem2):
        wid = lax.axis_index("s") * 2 + lax.axis_index("c")

        def body(ch, _):
            pltpu.sync_copy(src_hbm.at[wid, ch], idx_v)
            cp1 = pltpu.async_copy(qtab_hbm.at[idx_v], qrow_v, sem1)
            cp2 = pltpu.async_copy(auxtab_hbm.at[idx_v], arow_v, sem2)
            cp1.wait()
            cp2.wait()
            pltpu.sync_copy(qrow_v, qout_hbm.at[wid, ch])
            pltpu.sync_copy(arow_v, aout_hbm.at[wid, ch])
            return 0

        lax.fori_loop(0, _NCH_SC, body, 0)

    return gather_kernel(qtab, auxtab, src)


def _stage1_body(qg_ref, p_ref, st_ref):
    i = pl.program_id(0)
    p = p_ref[...]
    s = jnp.zeros((1, 128), jnp.float32)
    ss = jnp.zeros((1, 128), jnp.float32)
    for k in range(_K):
        h = qg_ref[k] + p
        s = s + jnp.sum(h, axis=0, keepdims=True)
        ss = ss + jnp.sum(h * h, axis=0, keepdims=True)

    @pl.when(i == 0)
    def _():
        st_ref[...] = jnp.zeros_like(st_ref)

    st_ref[0:1, :] = st_ref[0:1, :] + s
    st_ref[1:2, :] = st_ref[1:2, :] + ss


def _stage1(qg, p):
    return pl.pallas_call(
        _stage1_body,
        grid=(_NBLK,),
        in_specs=[
            pl.BlockSpec((_K, _BN, 128), lambda i: (0, i, 0)),
            pl.BlockSpec((_BN, 128), lambda i: (i, 0)),
        ],
        out_specs=pl.BlockSpec((8, 128), lambda i: (0, 0)),
        out_shape=jax.ShapeDtypeStruct((8, 128), jnp.float32),
    )(qg, p)


def _stage2_body(qg_ref, p_ref, st1_ref, w2_ref, b2_ref, g1_ref, bb1_ref,
                 h2_ref, st2_ref):
    i = pl.program_id(0)
    inv_e = 1.0 / float(_E)
    m = st1_ref[0:1, :] * inv_e
    v = st1_ref[1:2, :] * inv_e - m * m
    sc = g1_ref[...] / jnp.sqrt(v + 1e-5)
    sh = bb1_ref[...] - m * sc
    p = p_ref[...]
    hcat = jnp.concatenate([qg_ref[k] + p for k in range(_K)], axis=0)
    h = jnp.maximum(hcat * sc + sh, 0.0)
    h2 = jnp.dot(h, w2_ref[...], preferred_element_type=jnp.float32) \
        + b2_ref[...]
    s = jnp.sum(h2, axis=0, keepdims=True)
    ss = jnp.sum(h2 * h2, axis=0, keepdims=True)
    for k in range(_K):
        h2_ref[k] = h2[k * _BN:(k + 1) * _BN, :]

    @pl.when(i == 0)
    def _():
        st2_ref[...] = jnp.zeros_like(st2_ref)

    st2_ref[0:1, :] = st2_ref[0:1, :] + s
    st2_ref[1:2, :] = st2_ref[1:2, :] + ss


def _stage2(qg, p, st1, W2, b2r, g1r, bb1r):
    return pl.pallas_call(
        _stage2_body,
        grid=(_NBLK,),
        in_specs=[
            pl.BlockSpec((_K, _BN, 128), lambda i: (0, i, 0)),
            pl.BlockSpec((_BN, 128), lambda i: (i, 0)),
            pl.BlockSpec((8, 128), lambda i: (0, 0)),
            pl.BlockSpec((128, 128), lambda i: (0, 0)),
            pl.BlockSpec((1, 128), lambda i: (0, 0)),
            pl.BlockSpec((1, 128), lambda i: (0, 0)),
            pl.BlockSpec((1, 128), lambda i: (0, 0)),
        ],
        out_specs=(pl.BlockSpec((_K, _BN, 128), lambda i: (0, i, 0)),
                   pl.BlockSpec((8, 128), lambda i: (0, 0))),
        out_shape=(jax.ShapeDtypeStruct((_K, _NP, 128), jnp.float32),
                   jax.ShapeDtypeStruct((8, 128), jnp.float32)),
    )(qg, p, st1, W2, b2r, g1r, bb1r)


def _stage3_body(h2_ref, ag_ref, aux_ref, st2_ref, wa1_ref, ba1_ref, wa2_ref,
                 ba2_ref, g2_ref, bb2_ref, lng_ref, lnb_ref, out_ref):
    inv_e = 1.0 / float(_E)
    m = st2_ref[0:1, :] * inv_e
    v = st2_ref[1:2, :] * inv_e - m * m
    sc2 = g2_ref[...] / jnp.sqrt(v + 1e-5)
    sh2 = bb2_ref[...] - m * sc2
    ai = aux_ref[...]
    ea = jnp.concatenate(
        [jnp.concatenate([ai, ag_ref[k][:, 0:16]], axis=1) for k in range(_K)],
        axis=0)
    a = jnp.maximum(
        jnp.dot(ea, wa1_ref[...], preferred_element_type=jnp.float32)
        + ba1_ref[...], 0.0)
    gb = jnp.dot(a, wa2_ref[...], preferred_element_type=jnp.float32) \
        + ba2_ref[...]
    o = jnp.full((_BN, 128), -jnp.inf, jnp.float32)
    for k in range(_K):
        ef = jnp.maximum(h2_ref[k] * sc2 + sh2, 0.0)
        gbk = gb[k * _BN:(k + 1) * _BN, :]
        gamma = 1.0 / (1.0 + jnp.exp(-(gbk[:, 0:128] + 1.0)))
        beta = gbk[:, 128:256]
        o = jnp.maximum(o, gamma * ef + beta)
    mu = jnp.mean(o, axis=1, keepdims=True)
    d = o - mu
    var = jnp.mean(d * d, axis=1, keepdims=True)
    out_ref[...] = jnp.maximum(
        d / jnp.sqrt(var + 1e-5) * lng_ref[...] + lnb_ref[...], 0.0)


def _stage3(h2, ag, aux, st2, Wa1, ba1r, Wa2, ba2r, g2r, bb2r, lngr, lnbr):
    return pl.pallas_call(
        _stage3_body,
        grid=(_NBLK,),
        in_specs=[
            pl.BlockSpec((_K, _BN, 128), lambda i: (0, i, 0)),
            pl.BlockSpec((_K, _BN, 128), lambda i: (0, i, 0)),
            pl.BlockSpec((_BN, 16), lambda i: (i, 0)),
            pl.BlockSpec((8, 128), lambda i: (0, 0)),
            pl.BlockSpec((32, 64), lambda i: (0, 0)),
            pl.BlockSpec((1, 64), lambda i: (0, 0)),
            pl.BlockSpec((64, 256), lambda i: (0, 0)),
            pl.BlockSpec((1, 256), lambda i: (0, 0)),
            pl.BlockSpec((1, 128), lambda i: (0, 0)),
            pl.BlockSpec((1, 128), lambda i: (0, 0)),
            pl.BlockSpec((1, 128), lambda i: (0, 0)),
            pl.BlockSpec((1, 128), lambda i: (0, 0)),
        ],
        out_specs=pl.BlockSpec((_BN, 128), lambda i: (i, 0)),
        out_shape=jax.ShapeDtypeStruct((_N, 128), jnp.float32),
    )(h2, ag, aux, st2, Wa1, ba1r, Wa2, ba2r, g2r, bb2r, lngr, lnbr)


def kernel(geom, aux, batch, W1, b1, bn1_g, bn1_b, W2, b2, bn2_g, bn2_b,
           Wa1, ba1, Wa2, ba2, ln_g, ln_b):
    batch = batch.astype(jnp.int32)
    geom_pad = jnp.pad(geom, ((0, _NP - _N), (0, 0)))
    gt = geom_pad.T.reshape(128, _NCHUNK, _C).transpose(1, 0, 2)
    brow = jnp.pad(batch, (0, _NP - _N),
                   constant_values=-1).reshape(_NP, 1)
    bcol = jnp.pad(batch, (0, _NP - _N),
                   constant_values=-2).reshape(_NCHUNK, 1, _C)

    nb = _NP // _RB
    first_idx = jnp.minimum(jnp.arange(nb) * _RB, _N - 1)
    last_idx = jnp.minimum((jnp.arange(nb) + 1) * _RB - 1, _N - 1)
    lo = jnp.searchsorted(batch, batch[first_idx],
                          side='left').astype(jnp.int32)
    hi = jnp.searchsorted(batch, batch[last_idx],
                          side='right').astype(jnp.int32)

    nbr = _knn(lo, hi, geom_pad, gt, brow, bcol)
    src = jnp.pad(nbr[:_N, :_K].T, ((0, 0), (0, _NP - _N)))
    src = src.reshape(_NW, _NCH_SC, _CB)

    b1r = b1.reshape(1, 128)
    p, q = _prep(geom, W1, b1r)

    aux_pad = jnp.pad(aux, ((0, 0), (0, 112)))
    qg4, ag4 = _sc_gather_call(q, aux_pad, src)
    qg = qg4.reshape(_K, _NP, 128)
    ag = ag4.reshape(_K, _NP, 128)

    st1 = _stage1(qg, p)
    h2, st2 = _stage2(qg, p, st1, W2, b2.reshape(1, 128),
                      bn1_g.reshape(1, 128), bn1_b.reshape(1, 128))
    out = _stage3(h2, ag, aux, st2, Wa1, ba1.reshape(1, 64), Wa2,
                  ba2.reshape(1, 256), bn2_g.reshape(1, 128),
                  bn2_b.reshape(1, 128), ln_g.reshape(1, 128),
                  ln_b.reshape(1, 128))
    return out


# speedup vs baseline: 5.7723x; 1.2009x over previous
import functools

import jax
import jax.numpy as jnp
from jax import lax
from jax.experimental import pallas as pl
from jax.experimental.pallas import tpu as pltpu
from jax.experimental.pallas import tpu_sc as plsc

_N = 10000
_NP = 10240
_K = 20
_C = 512
_NCHUNK = _NP // _C
_RB = 256
_BN = 400
_NBLK = _N // _BN
_E = _N * _K
_NW = 32
_CB = 128
_NCH_SC = (_K * _NP) // (_NW * _CB)

_BIG2 = 3.0e38
_BIGF = 1.0e9


def _knn_body(lo_ref, hi_ref, g_ref, gt_ref, brow_ref, bcol_ref, nbr_ref,
              dist_ref):
    i = pl.program_id(0)
    g = g_ref[...]
    sq_r = jnp.sum(g * g, axis=1, keepdims=True)
    row_ids = i * _RB + lax.broadcasted_iota(jnp.int32, (_RB, 1), 0)
    brow = brow_ref[...]
    lo = lo_ref[i]
    hi = hi_ref[i]
    clo = lo // _C
    chi = (hi + _C - 1) // _C

    def dist_body(c, _):
        gt = gt_ref[c]
        d = -2.0 * jnp.dot(g, gt, preferred_element_type=jnp.float32)
        sq_c = jnp.sum(gt * gt, axis=0, keepdims=True)
        d = d + sq_r + sq_c
        bcol = bcol_ref[c]
        col_ids = c * _C + lax.broadcasted_iota(jnp.int32, (_RB, _C), 1)
        bad = (brow != bcol) | (col_ids == row_ids)
        dist_ref[c] = jnp.where(bad, jnp.inf, d)
        return 0

    lax.fori_loop(clo, chi, dist_body, 0)

    colf_base = lax.broadcasted_iota(jnp.int32, (_RB, _C),
                                     1).astype(jnp.float32)
    m_prev = jnp.full((_RB, 1), -jnp.inf, jnp.float32)
    i_prev = jnp.full((_RB, 1), -1.0, jnp.float32)
    for k in range(_K):
        def sel_body(c, carry, m_prev=m_prev, i_prev=i_prev):
            bv, bi = carry
            d = dist_ref[c]
            colf = (c * _C).astype(jnp.float32) + colf_base
            valid = (d > m_prev) | ((d == m_prev) & (colf > i_prev))
            dm = jnp.where(valid, d, _BIG2)
            cmin = jnp.min(dm, axis=1, keepdims=True)
            cidx = jnp.min(jnp.where(dm == cmin, colf, _BIGF),
                           axis=1, keepdims=True)
            take = (cmin < bv) | ((cmin == bv) & (cidx < bi))
            return jnp.where(take, cmin, bv), jnp.where(take, cidx, bi)

        best_v, best_i = lax.fori_loop(
            clo, chi, sel_body,
            (jnp.full((_RB, 1), _BIG2, jnp.float32),
             jnp.full((_RB, 1), -1.0, jnp.float32)))
        nbr_ref[:, k:k + 1] = jnp.clip(best_i, 0.0,
                                       float(_N - 1)).astype(jnp.int32)
        m_prev, i_prev = best_v, best_i


def _knn(lo, hi, geom_pad, gt, brow, bcol):
    return pl.pallas_call(
        _knn_body,
        grid=(_NP // _RB,),
        in_specs=[
            pl.BlockSpec(memory_space=pltpu.SMEM),
            pl.BlockSpec(memory_space=pltpu.SMEM),
            pl.BlockSpec((_RB, 128), lambda i: (i, 0)),
            pl.BlockSpec((_NCHUNK, 128, _C), lambda i: (0, 0, 0)),
            pl.BlockSpec((_RB, 1), lambda i: (i, 0)),
            pl.BlockSpec((_NCHUNK, 1, _C), lambda i: (0, 0, 0)),
        ],
        out_specs=pl.BlockSpec((_RB, 128), lambda i: (i, 0)),
        out_shape=jax.ShapeDtypeStruct((_NP, 128), jnp.int32),
        scratch_shapes=[pltpu.VMEM((_NCHUNK, _RB, _C), jnp.float32)],
    )(lo, hi, geom_pad, gt, brow, bcol)


def _prep_body(g_ref, aux_ref, w1_ref, b1_ref, p_ref, tab_ref):
    wa = w1_ref[0:128, :]
    wb = w1_ref[128:256, :]
    g = g_ref[...]
    p_ref[...] = (jnp.dot(g, wa - wb, preferred_element_type=jnp.float32)
                  + b1_ref[...])
    q = jnp.dot(g, wb, preferred_element_type=jnp.float32)
    tab_ref[...] = jnp.concatenate(
        [q, aux_ref[...], jnp.zeros((_N, 112), jnp.float32)], axis=1)


def _prep(geom, aux, W1, b1r):
    return pl.pallas_call(
        _prep_body,
        out_shape=(jax.ShapeDtypeStruct((_N, 128), jnp.float32),
                   jax.ShapeDtypeStruct((_N, 256), jnp.float32)),
    )(geom, aux, W1, b1r)


def _sc_gather_call(tab, src):
    mesh = plsc.VectorSubcoreMesh(core_axis_name="c", subcore_axis_name="s")

    @functools.partial(
        pl.kernel,
        mesh=mesh,
        out_type=jax.ShapeDtypeStruct((_NW, _NCH_SC, _CB, 256), jnp.float32),
        scratch_types=[
            pltpu.VMEM((_NCH_SC, _CB), jnp.int32),
            pltpu.VMEM((_CB, 256), jnp.float32),
            pltpu.VMEM((_CB, 256), jnp.float32),
            pltpu.SemaphoreType.DMA,
            pltpu.SemaphoreType.DMA,
        ],
    )
    def gather_kernel(tab_hbm, src_hbm, out_hbm, idx_v, buf0, buf1, sem0,
                      sem1):
        wid = lax.axis_index("s") * 2 + lax.axis_index("c")
        pltpu.sync_copy(src_hbm.at[wid], idx_v)
        bufs = (buf0, buf1)
        sems = (sem0, sem1)
        pltpu.async_copy(tab_hbm.at[idx_v.at[0]], buf0, sem0)

        @pl.loop(0, _NCH_SC, step=2)
        def _(c):
            for b in range(2):
                ch = c + b
                nxt = ch + 1
                pltpu.make_async_copy(tab_hbm.at[idx_v.at[ch % _NCH_SC]],
                                      bufs[b], sems[b]).wait()

                @pl.when(nxt < _NCH_SC)
                def _():
                    pltpu.async_copy(tab_hbm.at[idx_v.at[nxt % _NCH_SC]],
                                     bufs[1 - b], sems[1 - b])

                pltpu.sync_copy(bufs[b], out_hbm.at[wid, ch])

    return gather_kernel(tab, src)


def _stage1_body(qg_ref, p_ref, st_ref):
    i = pl.program_id(0)
    p = p_ref[...]
    s = jnp.zeros((1, 128), jnp.float32)
    ss = jnp.zeros((1, 128), jnp.float32)
    for k in range(_K):
        h = qg_ref[k] + p
        s = s + jnp.sum(h, axis=0, keepdims=True)
        ss = ss + jnp.sum(h * h, axis=0, keepdims=True)

    @pl.when(i == 0)
    def _():
        st_ref[...] = jnp.zeros_like(st_ref)

    st_ref[0:1, :] = st_ref[0:1, :] + s
    st_ref[1:2, :] = st_ref[1:2, :] + ss


def _stage1(qg, p):
    return pl.pallas_call(
        _stage1_body,
        grid=(_NBLK,),
        in_specs=[
            pl.BlockSpec((_K, _BN, 128), lambda i: (0, i, 0)),
            pl.BlockSpec((_BN, 128), lambda i: (i, 0)),
        ],
        out_specs=pl.BlockSpec((8, 128), lambda i: (0, 0)),
        out_shape=jax.ShapeDtypeStruct((8, 128), jnp.float32),
    )(qg, p)


def _stage2_body(qg_ref, p_ref, st1_ref, w2_ref, b2_ref, g1_ref, bb1_ref,
                 h2_ref, st2_ref):
    i = pl.program_id(0)
    inv_e = 1.0 / float(_E)
    m = st1_ref[0:1, :] * inv_e
    v = st1_ref[1:2, :] * inv_e - m * m
    sc = g1_ref[...] / jnp.sqrt(v + 1e-5)
    sh = bb1_ref[...] - m * sc
    p = p_ref[...]
    hcat = jnp.concatenate([qg_ref[k] + p for k in range(_K)], axis=0)
    h = jnp.maximum(hcat * sc + sh, 0.0)
    h2 = jnp.dot(h, w2_ref[...], preferred_element_type=jnp.float32) \
        + b2_ref[...]
    s = jnp.sum(h2, axis=0, keepdims=True)
    ss = jnp.sum(h2 * h2, axis=0, keepdims=True)
    for k in range(_K):
        h2_ref[k] = h2[k * _BN:(k + 1) * _BN, :]

    @pl.when(i == 0)
    def _():
        st2_ref[...] = jnp.zeros_like(st2_ref)

    st2_ref[0:1, :] = st2_ref[0:1, :] + s
    st2_ref[1:2, :] = st2_ref[1:2, :] + ss


def _stage2(qg, p, st1, W2, b2r, g1r, bb1r):
    return pl.pallas_call(
        _stage2_body,
        grid=(_NBLK,),
        in_specs=[
            pl.BlockSpec((_K, _BN, 128), lambda i: (0, i, 0)),
            pl.BlockSpec((_BN, 128), lambda i: (i, 0)),
            pl.BlockSpec((8, 128), lambda i: (0, 0)),
            pl.BlockSpec((128, 128), lambda i: (0, 0)),
            pl.BlockSpec((1, 128), lambda i: (0, 0)),
            pl.BlockSpec((1, 128), lambda i: (0, 0)),
            pl.BlockSpec((1, 128), lambda i: (0, 0)),
        ],
        out_specs=(pl.BlockSpec((_K, _BN, 128), lambda i: (0, i, 0)),
                   pl.BlockSpec((8, 128), lambda i: (0, 0))),
        out_shape=(jax.ShapeDtypeStruct((_K, _NP, 128), jnp.float32),
                   jax.ShapeDtypeStruct((8, 128), jnp.float32)),
    )(qg, p, st1, W2, b2r, g1r, bb1r)


def _stage3_body(h2_ref, ag_ref, aux_ref, st2_ref, wa1_ref, ba1_ref, wa2_ref,
                 ba2_ref, g2_ref, bb2_ref, lng_ref, lnb_ref, out_ref):
    inv_e = 1.0 / float(_E)
    m = st2_ref[0:1, :] * inv_e
    v = st2_ref[1:2, :] * inv_e - m * m
    sc2 = g2_ref[...] / jnp.sqrt(v + 1e-5)
    sh2 = bb2_ref[...] - m * sc2
    ai = aux_ref[...]
    ea = jnp.concatenate(
        [jnp.concatenate([ai, ag_ref[k][:, 0:16]], axis=1) for k in range(_K)],
        axis=0)
    a = jnp.maximum(
        jnp.dot(ea, wa1_ref[...], preferred_element_type=jnp.float32)
        + ba1_ref[...], 0.0)
    gb = jnp.dot(a, wa2_ref[...], preferred_element_type=jnp.float32) \
        + ba2_ref[...]
    o = jnp.full((_BN, 128), -jnp.inf, jnp.float32)
    for k in range(_K):
        ef = jnp.maximum(h2_ref[k] * sc2 + sh2, 0.0)
        gbk = gb[k * _BN:(k + 1) * _BN, :]
        gamma = 1.0 / (1.0 + jnp.exp(-(gbk[:, 0:128] + 1.0)))
        beta = gbk[:, 128:256]
        o = jnp.maximum(o, gamma * ef + beta)
    mu = jnp.mean(o, axis=1, keepdims=True)
    d = o - mu
    var = jnp.mean(d * d, axis=1, keepdims=True)
    out_ref[...] = jnp.maximum(
        d / jnp.sqrt(var + 1e-5) * lng_ref[...] + lnb_ref[...], 0.0)


def _stage3(h2, ag, aux, st2, Wa1, ba1r, Wa2, ba2r, g2r, bb2r, lngr, lnbr):
    return pl.pallas_call(
        _stage3_body,
        grid=(_NBLK,),
        in_specs=[
            pl.BlockSpec((_K, _BN, 128), lambda i: (0, i, 0)),
            pl.BlockSpec((_K, _BN, 128), lambda i: (0, i, 1)),
            pl.BlockSpec((_BN, 16), lambda i: (i, 0)),
            pl.BlockSpec((8, 128), lambda i: (0, 0)),
            pl.BlockSpec((32, 64), lambda i: (0, 0)),
            pl.BlockSpec((1, 64), lambda i: (0, 0)),
            pl.BlockSpec((64, 256), lambda i: (0, 0)),
            pl.BlockSpec((1, 256), lambda i: (0, 0)),
            pl.BlockSpec((1, 128), lambda i: (0, 0)),
            pl.BlockSpec((1, 128), lambda i: (0, 0)),
            pl.BlockSpec((1, 128), lambda i: (0, 0)),
            pl.BlockSpec((1, 128), lambda i: (0, 0)),
        ],
        out_specs=pl.BlockSpec((_BN, 128), lambda i: (i, 0)),
        out_shape=jax.ShapeDtypeStruct((_N, 128), jnp.float32),
    )(h2, ag, aux, st2, Wa1, ba1r, Wa2, ba2r, g2r, bb2r, lngr, lnbr)


def kernel(geom, aux, batch, W1, b1, bn1_g, bn1_b, W2, b2, bn2_g, bn2_b,
           Wa1, ba1, Wa2, ba2, ln_g, ln_b):
    batch = batch.astype(jnp.int32)
    geom_pad = jnp.pad(geom, ((0, _NP - _N), (0, 0)))
    gt = geom_pad.T.reshape(128, _NCHUNK, _C).transpose(1, 0, 2)
    brow = jnp.pad(batch, (0, _NP - _N),
                   constant_values=-1).reshape(_NP, 1)
    bcol = jnp.pad(batch, (0, _NP - _N),
                   constant_values=-2).reshape(_NCHUNK, 1, _C)

    nb = _NP // _RB
    first_idx = jnp.minimum(jnp.arange(nb) * _RB, _N - 1)
    last_idx = jnp.minimum((jnp.arange(nb) + 1) * _RB - 1, _N - 1)
    lo = jnp.searchsorted(batch, batch[first_idx],
                          side='left').astype(jnp.int32)
    hi = jnp.searchsorted(batch, batch[last_idx],
                          side='right').astype(jnp.int32)

    nbr = _knn(lo, hi, geom_pad, gt, brow, bcol)
    src = jnp.pad(nbr[:_N, :_K].T, ((0, 0), (0, _NP - _N)))
    src = src.reshape(_NW, _NCH_SC, _CB)

    b1r = b1.reshape(1, 128)
    p, tab = _prep(geom, aux, W1, b1r)

    eg4 = _sc_gather_call(tab, src)
    eg = eg4.reshape(_K, _NP, 256)

    st1 = _stage1(eg, p)
    h2, st2 = _stage2(eg, p, st1, W2, b2.reshape(1, 128),
                      bn1_g.reshape(1, 128), bn1_b.reshape(1, 128))
    out = _stage3(h2, eg, aux, st2, Wa1, ba1.reshape(1, 64), Wa2,
                  ba2.reshape(1, 256), bn2_g.reshape(1, 128),
                  bn2_b.reshape(1, 128), ln_g.reshape(1, 128),
                  ln_b.reshape(1, 128))
    return out


# speedup vs baseline: 6.6983x; 1.1604x over previous
import functools

import jax
import jax.numpy as jnp
from jax import lax
from jax.experimental import pallas as pl
from jax.experimental.pallas import tpu as pltpu
from jax.experimental.pallas import tpu_sc as plsc

_N = 10000
_NP = 10240
_K = 20
_C = 512
_NCHUNK = _NP // _C
_RB = 256
_BN = 400
_NBLK = _N // _BN
_E = _N * _K
_NW = 32
_CB = 128
_NCH_SC = (_K * _NP) // (_NW * _CB)

_BIG2 = 3.0e38
_BIGF = 1.0e9


def _knn_body(lo_ref, hi_ref, g_ref, gt_ref, brow_ref, bcol_ref, nbr_ref,
              dist_ref):
    i = pl.program_id(0)
    g = g_ref[...]
    row_ids = i * _RB + lax.broadcasted_iota(jnp.int32, (_RB, 1), 0)
    brow = brow_ref[...]
    lo = lo_ref[i]
    hi = hi_ref[i]
    clo = lo // _C
    chi = (hi + _C - 1) // _C

    def dist_body(c, _):
        gt = gt_ref[c]
        d = -2.0 * jnp.dot(g, gt, preferred_element_type=jnp.float32)
        sq_c = jnp.sum(gt * gt, axis=0, keepdims=True)
        d = d + sq_c
        bcol = bcol_ref[c]
        col_ids = c * _C + lax.broadcasted_iota(jnp.int32, (_RB, _C), 1)
        bad = (brow != bcol) | (col_ids == row_ids)
        dist_ref[c] = jnp.where(bad, jnp.inf, d)
        return 0

    lax.fori_loop(clo, chi, dist_body, 0)

    colf_base = lax.broadcasted_iota(jnp.int32, (_RB, _C),
                                     1).astype(jnp.float32)
    i_prev = jnp.full((_RB, 1), -1.0, jnp.float32)
    for k in range(_K):
        def sel_body(c, carry, i_prev=i_prev):
            bv, bi = carry
            colf = (c * _C).astype(jnp.float32) + colf_base
            d = jnp.where(colf == i_prev, _BIG2, dist_ref[c])
            dist_ref[c] = d
            cmin = jnp.min(d, axis=1, keepdims=True)
            cidx = jnp.min(jnp.where(d == cmin, colf, _BIGF),
                           axis=1, keepdims=True)
            take = (cmin < bv) | ((cmin == bv) & (cidx < bi))
            return jnp.where(take, cmin, bv), jnp.where(take, cidx, bi)

        best_v, best_i = lax.fori_loop(
            clo, chi, sel_body,
            (jnp.full((_RB, 1), _BIG2, jnp.float32),
             jnp.full((_RB, 1), -1.0, jnp.float32)))
        nbr_ref[:, k:k + 1] = jnp.clip(best_i, 0.0,
                                       float(_N - 1)).astype(jnp.int32)
        i_prev = best_i


def _knn(lo, hi, geom_pad, gt, brow, bcol):
    return pl.pallas_call(
        _knn_body,
        grid=(_NP // _RB,),
        in_specs=[
            pl.BlockSpec(memory_space=pltpu.SMEM),
            pl.BlockSpec(memory_space=pltpu.SMEM),
            pl.BlockSpec((_RB, 128), lambda i: (i, 0)),
            pl.BlockSpec((_NCHUNK, 128, _C), lambda i: (0, 0, 0)),
            pl.BlockSpec((_RB, 1), lambda i: (i, 0)),
            pl.BlockSpec((_NCHUNK, 1, _C), lambda i: (0, 0, 0)),
        ],
        out_specs=pl.BlockSpec((_RB, 128), lambda i: (i, 0)),
        out_shape=jax.ShapeDtypeStruct((_NP, 128), jnp.int32),
        scratch_shapes=[pltpu.VMEM((_NCHUNK, _RB, _C), jnp.float32)],
    )(lo, hi, geom_pad, gt, brow, bcol)


def _prep_body(g_ref, aux_ref, w1_ref, b1_ref, p_ref, tab_ref):
    wa = w1_ref[0:128, :]
    wb = w1_ref[128:256, :]
    g = g_ref[...]
    p_ref[...] = (jnp.dot(g, wa - wb, preferred_element_type=jnp.float32)
                  + b1_ref[...])
    q = jnp.dot(g, wb, preferred_element_type=jnp.float32)
    tab_ref[...] = jnp.concatenate(
        [q, aux_ref[...], jnp.zeros((_N, 112), jnp.float32)], axis=1)


def _prep(geom, aux, W1, b1r):
    return pl.pallas_call(
        _prep_body,
        out_shape=(jax.ShapeDtypeStruct((_N, 128), jnp.float32),
                   jax.ShapeDtypeStruct((_N, 256), jnp.float32)),
    )(geom, aux, W1, b1r)


def _sc_gather_call(tab, src):
    mesh = plsc.VectorSubcoreMesh(core_axis_name="c", subcore_axis_name="s")

    @functools.partial(
        pl.kernel,
        mesh=mesh,
        out_type=jax.ShapeDtypeStruct((_NW, _NCH_SC, _CB, 256), jnp.float32),
        scratch_types=[
            pltpu.VMEM((_NCH_SC, _CB), jnp.int32),
            pltpu.VMEM((_CB, 256), jnp.float32),
            pltpu.VMEM((_CB, 256), jnp.float32),
            pltpu.SemaphoreType.DMA,
            pltpu.SemaphoreType.DMA,
        ],
    )
    def gather_kernel(tab_hbm, src_hbm, out_hbm, idx_v, buf0, buf1, sem0,
                      sem1):
        wid = lax.axis_index("s") * 2 + lax.axis_index("c")
        pltpu.sync_copy(src_hbm.at[wid], idx_v)
        bufs = (buf0, buf1)
        sems = (sem0, sem1)
        pltpu.async_copy(tab_hbm.at[idx_v.at[0]], buf0, sem0)

        @pl.loop(0, _NCH_SC, step=2)
        def _(c):
            for b in range(2):
                ch = c + b
                nxt = ch + 1
                pltpu.make_async_copy(tab_hbm.at[idx_v.at[ch % _NCH_SC]],
                                      bufs[b], sems[b]).wait()

                @pl.when(nxt < _NCH_SC)
                def _():
                    pltpu.async_copy(tab_hbm.at[idx_v.at[nxt % _NCH_SC]],
                                     bufs[1 - b], sems[1 - b])

                pltpu.sync_copy(bufs[b], out_hbm.at[wid, ch])

    return gather_kernel(tab, src)


def _stage1_body(qg_ref, p_ref, st_ref):
    i = pl.program_id(0)
    p = p_ref[...]
    s = jnp.zeros((1, 128), jnp.float32)
    ss = jnp.zeros((1, 128), jnp.float32)
    for k in range(_K):
        h = qg_ref[k] + p
        s = s + jnp.sum(h, axis=0, keepdims=True)
        ss = ss + jnp.sum(h * h, axis=0, keepdims=True)

    @pl.when(i == 0)
    def _():
        st_ref[...] = jnp.zeros_like(st_ref)

    st_ref[0:1, :] = st_ref[0:1, :] + s
    st_ref[1:2, :] = st_ref[1:2, :] + ss


def _stage1(qg, p):
    return pl.pallas_call(
        _stage1_body,
        grid=(_NBLK,),
        in_specs=[
            pl.BlockSpec((_K, _BN, 128), lambda i: (0, i, 0)),
            pl.BlockSpec((_BN, 128), lambda i: (i, 0)),
        ],
        out_specs=pl.BlockSpec((8, 128), lambda i: (0, 0)),
        out_shape=jax.ShapeDtypeStruct((8, 128), jnp.float32),
    )(qg, p)


def _stage2_body(qg_ref, p_ref, st1_ref, w2_ref, b2_ref, g1_ref, bb1_ref,
                 h2_ref, st2_ref):
    i = pl.program_id(0)
    inv_e = 1.0 / float(_E)
    m = st1_ref[0:1, :] * inv_e
    v = st1_ref[1:2, :] * inv_e - m * m
    sc = g1_ref[...] / jnp.sqrt(v + 1e-5)
    sh = bb1_ref[...] - m * sc
    p = p_ref[...]
    hcat = jnp.concatenate([qg_ref[k] + p for k in range(_K)], axis=0)
    h = jnp.maximum(hcat * sc + sh, 0.0)
    h2 = jnp.dot(h, w2_ref[...], preferred_element_type=jnp.float32) \
        + b2_ref[...]
    s = jnp.sum(h2, axis=0, keepdims=True)
    ss = jnp.sum(h2 * h2, axis=0, keepdims=True)
    for k in range(_K):
        h2_ref[k] = h2[k * _BN:(k + 1) * _BN, :]

    @pl.when(i == 0)
    def _():
        st2_ref[...] = jnp.zeros_like(st2_ref)

    st2_ref[0:1, :] = st2_ref[0:1, :] + s
    st2_ref[1:2, :] = st2_ref[1:2, :] + ss


def _stage2(qg, p, st1, W2, b2r, g1r, bb1r):
    return pl.pallas_call(
        _stage2_body,
        grid=(_NBLK,),
        in_specs=[
            pl.BlockSpec((_K, _BN, 128), lambda i: (0, i, 0)),
            pl.BlockSpec((_BN, 128), lambda i: (i, 0)),
            pl.BlockSpec((8, 128), lambda i: (0, 0)),
            pl.BlockSpec((128, 128), lambda i: (0, 0)),
            pl.BlockSpec((1, 128), lambda i: (0, 0)),
            pl.BlockSpec((1, 128), lambda i: (0, 0)),
            pl.BlockSpec((1, 128), lambda i: (0, 0)),
        ],
        out_specs=(pl.BlockSpec((_K, _BN, 128), lambda i: (0, i, 0)),
                   pl.BlockSpec((8, 128), lambda i: (0, 0))),
        out_shape=(jax.ShapeDtypeStruct((_K, _NP, 128), jnp.float32),
                   jax.ShapeDtypeStruct((8, 128), jnp.float32)),
    )(qg, p, st1, W2, b2r, g1r, bb1r)


def _stage3_body(h2_ref, ag_ref, aux_ref, st2_ref, wa1_ref, ba1_ref, wa2_ref,
                 ba2_ref, g2_ref, bb2_ref, lng_ref, lnb_ref, out_ref):
    inv_e = 1.0 / float(_E)
    m = st2_ref[0:1, :] * inv_e
    v = st2_ref[1:2, :] * inv_e - m * m
    sc2 = g2_ref[...] / jnp.sqrt(v + 1e-5)
    sh2 = bb2_ref[...] - m * sc2
    ai = aux_ref[...]
    ea = jnp.concatenate(
        [jnp.concatenate([ai, ag_ref[k][:, 0:16]], axis=1) for k in range(_K)],
        axis=0)
    a = jnp.maximum(
        jnp.dot(ea, wa1_ref[...], preferred_element_type=jnp.float32)
        + ba1_ref[...], 0.0)
    gb = jnp.dot(a, wa2_ref[...], preferred_element_type=jnp.float32) \
        + ba2_ref[...]
    o = jnp.full((_BN, 128), -jnp.inf, jnp.float32)
    for k in range(_K):
        ef = jnp.maximum(h2_ref[k] * sc2 + sh2, 0.0)
        gbk = gb[k * _BN:(k + 1) * _BN, :]
        gamma = 1.0 / (1.0 + jnp.exp(-(gbk[:, 0:128] + 1.0)))
        beta = gbk[:, 128:256]
        o = jnp.maximum(o, gamma * ef + beta)
    mu = jnp.mean(o, axis=1, keepdims=True)
    d = o - mu
    var = jnp.mean(d * d, axis=1, keepdims=True)
    out_ref[...] = jnp.maximum(
        d / jnp.sqrt(var + 1e-5) * lng_ref[...] + lnb_ref[...], 0.0)


def _stage3(h2, ag, aux, st2, Wa1, ba1r, Wa2, ba2r, g2r, bb2r, lngr, lnbr):
    return pl.pallas_call(
        _stage3_body,
        grid=(_NBLK,),
        in_specs=[
            pl.BlockSpec((_K, _BN, 128), lambda i: (0, i, 0)),
            pl.BlockSpec((_K, _BN, 128), lambda i: (0, i, 1)),
            pl.BlockSpec((_BN, 16), lambda i: (i, 0)),
            pl.BlockSpec((8, 128), lambda i: (0, 0)),
            pl.BlockSpec((32, 64), lambda i: (0, 0)),
            pl.BlockSpec((1, 64), lambda i: (0, 0)),
            pl.BlockSpec((64, 256), lambda i: (0, 0)),
            pl.BlockSpec((1, 256), lambda i: (0, 0)),
            pl.BlockSpec((1, 128), lambda i: (0, 0)),
            pl.BlockSpec((1, 128), lambda i: (0, 0)),
            pl.BlockSpec((1, 128), lambda i: (0, 0)),
            pl.BlockSpec((1, 128), lambda i: (0, 0)),
        ],
        out_specs=pl.BlockSpec((_BN, 128), lambda i: (i, 0)),
        out_shape=jax.ShapeDtypeStruct((_N, 128), jnp.float32),
    )(h2, ag, aux, st2, Wa1, ba1r, Wa2, ba2r, g2r, bb2r, lngr, lnbr)


def kernel(geom, aux, batch, W1, b1, bn1_g, bn1_b, W2, b2, bn2_g, bn2_b,
           Wa1, ba1, Wa2, ba2, ln_g, ln_b):
    batch = batch.astype(jnp.int32)
    geom_pad = jnp.pad(geom, ((0, _NP - _N), (0, 0)))
    gt = geom_pad.T.reshape(128, _NCHUNK, _C).transpose(1, 0, 2)
    brow = jnp.pad(batch, (0, _NP - _N),
                   constant_values=-1).reshape(_NP, 1)
    bcol = jnp.pad(batch, (0, _NP - _N),
                   constant_values=-2).reshape(_NCHUNK, 1, _C)

    nb = _NP // _RB
    first_idx = jnp.minimum(jnp.arange(nb) * _RB, _N - 1)
    last_idx = jnp.minimum((jnp.arange(nb) + 1) * _RB - 1, _N - 1)
    lo = jnp.searchsorted(batch, batch[first_idx],
                          side='left').astype(jnp.int32)
    hi = jnp.searchsorted(batch, batch[last_idx],
                          side='right').astype(jnp.int32)

    nbr = _knn(lo, hi, geom_pad, gt, brow, bcol)
    src = jnp.pad(nbr[:_N, :_K].T, ((0, 0), (0, _NP - _N)))
    src = src.reshape(_NW, _NCH_SC, _CB)

    b1r = b1.reshape(1, 128)
    p, tab = _prep(geom, aux, W1, b1r)

    eg4 = _sc_gather_call(tab, src)
    eg = eg4.reshape(_K, _NP, 256)

    st1 = _stage1(eg, p)
    h2, st2 = _stage2(eg, p, st1, W2, b2.reshape(1, 128),
                      bn1_g.reshape(1, 128), bn1_b.reshape(1, 128))
    out = _stage3(h2, eg, aux, st2, Wa1, ba1.reshape(1, 64), Wa2,
                  ba2.reshape(1, 256), bn2_g.reshape(1, 128),
                  bn2_b.reshape(1, 128), ln_g.reshape(1, 128),
                  ln_b.reshape(1, 128))
    return out


# speedup vs baseline: 8.1160x; 1.2117x over previous
import functools

import jax
import jax.numpy as jnp
from jax import lax
from jax.experimental import pallas as pl
from jax.experimental.pallas import tpu as pltpu
from jax.experimental.pallas import tpu_sc as plsc

_N = 10000
_NP = 10240
_K = 20
_C = 512
_NCHUNK = _NP // _C
_RB = 256
_BN = 400
_NBLK = _N // _BN
_E = _N * _K
_NW = 32
_CB = 128
_NCH_SC = (_K * _NP) // (_NW * _CB)

_BIG2 = 3.0e38
_BIGF = 1.0e9


def _knn_body(lo_ref, hi_ref, g_ref, gt_ref, brow_ref, bcol_ref, nbr_ref,
              dist_ref):
    i = pl.program_id(0)
    g = g_ref[...]
    row_ids = i * _RB + lax.broadcasted_iota(jnp.int32, (_RB, 1), 0)
    brow = brow_ref[...]
    lo = lo_ref[i]
    hi = hi_ref[i]
    clo = lo // _C
    chi = (hi + _C - 1) // _C

    def dist_body(c, _):
        gt = gt_ref[c]
        d = -2.0 * jnp.dot(g, gt, preferred_element_type=jnp.float32)
        sq_c = jnp.sum(gt * gt, axis=0, keepdims=True)
        d = d + sq_c
        bcol = bcol_ref[c]
        col_ids = c * _C + lax.broadcasted_iota(jnp.int32, (_RB, _C), 1)
        bad = (brow != bcol) | (col_ids == row_ids)
        dist_ref[c] = jnp.where(bad, jnp.inf, d)
        return 0

    lax.fori_loop(clo, chi, dist_body, 0)

    colf_base = lax.broadcasted_iota(jnp.int32, (_RB, _C),
                                     1).astype(jnp.float32)
    i_prev = jnp.full((_RB, 1), -1.0, jnp.float32)
    for k in range(_K):
        def sel_body(c, carry, i_prev=i_prev):
            bv, bi = carry
            colf = (c * _C).astype(jnp.float32) + colf_base
            d = jnp.where(colf == i_prev, _BIG2, dist_ref[c])
            dist_ref[c] = d
            cmin = jnp.min(d, axis=1, keepdims=True)
            cidx = jnp.min(jnp.where(d == cmin, colf, _BIGF),
                           axis=1, keepdims=True)
            take = (cmin < bv) | ((cmin == bv) & (cidx < bi))
            return jnp.where(take, cmin, bv), jnp.where(take, cidx, bi)

        best_v, best_i = lax.fori_loop(
            clo, chi, sel_body,
            (jnp.full((_RB, 1), _BIG2, jnp.float32),
             jnp.full((_RB, 1), -1.0, jnp.float32)))
        nbr_ref[:, k:k + 1] = jnp.clip(best_i, 0.0,
                                       float(_N - 1)).astype(jnp.int32)
        i_prev = best_i


def _knn(lo, hi, geom_pad, gt, brow, bcol):
    return pl.pallas_call(
        _knn_body,
        grid=(_NP // _RB,),
        in_specs=[
            pl.BlockSpec(memory_space=pltpu.SMEM),
            pl.BlockSpec(memory_space=pltpu.SMEM),
            pl.BlockSpec((_RB, 128), lambda i: (i, 0)),
            pl.BlockSpec((_NCHUNK, 128, _C), lambda i: (0, 0, 0)),
            pl.BlockSpec((_RB, 1), lambda i: (i, 0)),
            pl.BlockSpec((_NCHUNK, 1, _C), lambda i: (0, 0, 0)),
        ],
        out_specs=pl.BlockSpec((_RB, 128), lambda i: (i, 0)),
        out_shape=jax.ShapeDtypeStruct((_NP, 128), jnp.int32),
        scratch_shapes=[pltpu.VMEM((_NCHUNK, _RB, _C), jnp.float32)],
    )(lo, hi, geom_pad, gt, brow, bcol)


def _prep_body(g_ref, aux_ref, w1_ref, b1_ref, p_ref, tab_ref):
    wa = w1_ref[0:128, :]
    wb = w1_ref[128:256, :]
    g = g_ref[...]
    p_ref[...] = (jnp.dot(g, wa - wb, preferred_element_type=jnp.float32)
                  + b1_ref[...])
    q = jnp.dot(g, wb, preferred_element_type=jnp.float32)
    tab_ref[...] = jnp.concatenate(
        [q, aux_ref[...], jnp.zeros((_N, 112), jnp.float32)], axis=1)


def _prep(geom, aux, W1, b1r):
    return pl.pallas_call(
        _prep_body,
        out_shape=(jax.ShapeDtypeStruct((_N, 128), jnp.float32),
                   jax.ShapeDtypeStruct((_N, 256), jnp.float32)),
    )(geom, aux, W1, b1r)


def _sc_gather_call(tab, src):
    mesh = plsc.VectorSubcoreMesh(core_axis_name="c", subcore_axis_name="s")

    @functools.partial(
        pl.kernel,
        mesh=mesh,
        out_type=jax.ShapeDtypeStruct((_NW, _NCH_SC, _CB, 256), jnp.float32),
        scratch_types=[
            pltpu.VMEM((_NCH_SC, _CB), jnp.int32),
            pltpu.VMEM((_CB, 256), jnp.float32),
            pltpu.VMEM((_CB, 256), jnp.float32),
            pltpu.SemaphoreType.DMA,
            pltpu.SemaphoreType.DMA,
        ],
    )
    def gather_kernel(tab_hbm, src_hbm, out_hbm, idx_v, buf0, buf1, sem0,
                      sem1):
        wid = lax.axis_index("s") * 2 + lax.axis_index("c")
        pltpu.sync_copy(src_hbm.at[wid], idx_v)
        bufs = (buf0, buf1)
        sems = (sem0, sem1)
        pltpu.async_copy(tab_hbm.at[idx_v.at[0]], buf0, sem0)

        @pl.loop(0, _NCH_SC, step=2)
        def _(c):
            for b in range(2):
                ch = c + b
                nxt = ch + 1
                pltpu.make_async_copy(tab_hbm.at[idx_v.at[ch % _NCH_SC]],
                                      bufs[b], sems[b]).wait()

                @pl.when(nxt < _NCH_SC)
                def _():
                    pltpu.async_copy(tab_hbm.at[idx_v.at[nxt % _NCH_SC]],
                                     bufs[1 - b], sems[1 - b])

                pltpu.sync_copy(bufs[b], out_hbm.at[wid, ch])

    return gather_kernel(tab, src)


def _stage1_body(qg_ref, p_ref, st_ref):
    i = pl.program_id(0)
    p = p_ref[...]
    s = jnp.zeros((1, 128), jnp.float32)
    ss = jnp.zeros((1, 128), jnp.float32)
    for k in range(_K):
        h = qg_ref[k] + p
        s = s + jnp.sum(h, axis=0, keepdims=True)
        ss = ss + jnp.sum(h * h, axis=0, keepdims=True)

    @pl.when(i == 0)
    def _():
        st_ref[...] = jnp.zeros_like(st_ref)

    st_ref[0:1, :] = st_ref[0:1, :] + s
    st_ref[1:2, :] = st_ref[1:2, :] + ss


def _stage1(qg, p):
    return pl.pallas_call(
        _stage1_body,
        grid=(_NBLK,),
        in_specs=[
            pl.BlockSpec((_K, _BN, 128), lambda i: (0, i, 0)),
            pl.BlockSpec((_BN, 128), lambda i: (i, 0)),
        ],
        out_specs=pl.BlockSpec((8, 128), lambda i: (0, 0)),
        out_shape=jax.ShapeDtypeStruct((8, 128), jnp.float32),
    )(qg, p)


def _stage2_body(qg_ref, p_ref, st1_ref, w2_ref, b2_ref, g1_ref, bb1_ref,
                 h2_ref, st2_ref):
    i = pl.program_id(0)
    inv_e = 1.0 / float(_E)
    m = st1_ref[0:1, :] * inv_e
    v = st1_ref[1:2, :] * inv_e - m * m
    sc = g1_ref[...] / jnp.sqrt(v + 1e-5)
    sh = bb1_ref[...] - m * sc
    p = p_ref[...]
    hcat = jnp.concatenate([qg_ref[k] + p for k in range(_K)], axis=0)
    h = jnp.maximum(hcat * sc + sh, 0.0)
    h2 = jnp.dot(h, w2_ref[...], preferred_element_type=jnp.float32) \
        + b2_ref[...]
    s = jnp.sum(h2, axis=0, keepdims=True)
    ss = jnp.sum(h2 * h2, axis=0, keepdims=True)
    for k in range(_K):
        h2_ref[k] = h2[k * _BN:(k + 1) * _BN, :]

    @pl.when(i == 0)
    def _():
        st2_ref[...] = jnp.zeros_like(st2_ref)

    st2_ref[0:1, :] = st2_ref[0:1, :] + s
    st2_ref[1:2, :] = st2_ref[1:2, :] + ss


def _stage2(qg, p, st1, W2, b2r, g1r, bb1r):
    return pl.pallas_call(
        _stage2_body,
        grid=(_NBLK,),
        in_specs=[
            pl.BlockSpec((_K, _BN, 128), lambda i: (0, i, 0)),
            pl.BlockSpec((_BN, 128), lambda i: (i, 0)),
            pl.BlockSpec((8, 128), lambda i: (0, 0)),
            pl.BlockSpec((128, 128), lambda i: (0, 0)),
            pl.BlockSpec((1, 128), lambda i: (0, 0)),
            pl.BlockSpec((1, 128), lambda i: (0, 0)),
            pl.BlockSpec((1, 128), lambda i: (0, 0)),
        ],
        out_specs=(pl.BlockSpec((_K, _BN, 128), lambda i: (0, i, 0)),
                   pl.BlockSpec((8, 128), lambda i: (0, 0))),
        out_shape=(jax.ShapeDtypeStruct((_K, _NP, 128), jnp.float32),
                   jax.ShapeDtypeStruct((8, 128), jnp.float32)),
    )(qg, p, st1, W2, b2r, g1r, bb1r)


def _stage3_body(h2_ref, ag_ref, aux_ref, st2_ref, wa1_ref, ba1_ref, wa2_ref,
                 ba2_ref, g2_ref, bb2_ref, lng_ref, lnb_ref, out_ref):
    inv_e = 1.0 / float(_E)
    m = st2_ref[0:1, :] * inv_e
    v = st2_ref[1:2, :] * inv_e - m * m
    sc2 = g2_ref[...] / jnp.sqrt(v + 1e-5)
    sh2 = bb2_ref[...] - m * sc2
    ai = aux_ref[...]
    ea = jnp.concatenate(
        [jnp.concatenate([ai, ag_ref[k][:, 0:16]], axis=1) for k in range(_K)],
        axis=0)
    a = jnp.maximum(
        jnp.dot(ea, wa1_ref[...], preferred_element_type=jnp.float32)
        + ba1_ref[...], 0.0)
    gb = jnp.dot(a, wa2_ref[...], preferred_element_type=jnp.float32) \
        + ba2_ref[...]
    o = jnp.full((_BN, 128), -jnp.inf, jnp.float32)
    for k in range(_K):
        ef = jnp.maximum(h2_ref[k] * sc2 + sh2, 0.0)
        gbk = gb[k * _BN:(k + 1) * _BN, :]
        gamma = 1.0 / (1.0 + jnp.exp(-(gbk[:, 0:128] + 1.0)))
        beta = gbk[:, 128:256]
        o = jnp.maximum(o, gamma * ef + beta)
    mu = jnp.mean(o, axis=1, keepdims=True)
    d = o - mu
    var = jnp.mean(d * d, axis=1, keepdims=True)
    out_ref[...] = jnp.maximum(
        d / jnp.sqrt(var + 1e-5) * lng_ref[...] + lnb_ref[...], 0.0)


def _stage3(h2, ag, aux, st2, Wa1, ba1r, Wa2, ba2r, g2r, bb2r, lngr, lnbr):
    return pl.pallas_call(
        _stage3_body,
        grid=(_NBLK,),
        in_specs=[
            pl.BlockSpec((_K, _BN, 128), lambda i: (0, i, 0)),
            pl.BlockSpec((_K, _BN, 128), lambda i: (0, i, 1)),
            pl.BlockSpec((_BN, 16), lambda i: (i, 0)),
            pl.BlockSpec((8, 128), lambda i: (0, 0)),
            pl.BlockSpec((32, 64), lambda i: (0, 0)),
            pl.BlockSpec((1, 64), lambda i: (0, 0)),
            pl.BlockSpec((64, 256), lambda i: (0, 0)),
            pl.BlockSpec((1, 256), lambda i: (0, 0)),
            pl.BlockSpec((1, 128), lambda i: (0, 0)),
            pl.BlockSpec((1, 128), lambda i: (0, 0)),
            pl.BlockSpec((1, 128), lambda i: (0, 0)),
            pl.BlockSpec((1, 128), lambda i: (0, 0)),
        ],
        out_specs=pl.BlockSpec((_BN, 128), lambda i: (i, 0)),
        out_shape=jax.ShapeDtypeStruct((_N, 128), jnp.float32),
    )(h2, ag, aux, st2, Wa1, ba1r, Wa2, ba2r, g2r, bb2r, lngr, lnbr)


def kernel(geom, aux, batch, W1, b1, bn1_g, bn1_b, W2, b2, bn2_g, bn2_b,
           Wa1, ba1, Wa2, ba2, ln_g, ln_b):
    batch = batch.astype(jnp.int32)
    geom_pad = jnp.pad(geom, ((0, _NP - _N), (0, 0)))
    gt = geom_pad.T.reshape(128, _NCHUNK, _C).transpose(1, 0, 2)
    brow = jnp.pad(batch, (0, _NP - _N),
                   constant_values=-1).reshape(_NP, 1)
    bcol = jnp.pad(batch, (0, _NP - _N),
                   constant_values=-2).reshape(_NCHUNK, 1, _C)

    nb = _NP // _RB
    first_idx = jnp.minimum(jnp.arange(nb) * _RB, _N - 1)
    last_idx = jnp.minimum((jnp.arange(nb) + 1) * _RB - 1, _N - 1)
    lo = jnp.searchsorted(batch, batch[first_idx],
                          side='left').astype(jnp.int32)
    hi = jnp.searchsorted(batch, batch[last_idx],
                          side='right').astype(jnp.int32)

    nbr = _knn(lo, hi, geom_pad, gt, brow, bcol)
    pad_idx = jnp.broadcast_to(
        jnp.arange(_NP - _N, dtype=jnp.int32)[None, :], (_K, _NP - _N))
    src = jnp.concatenate([nbr[:_N, :_K].T, pad_idx], axis=1)
    src = src.reshape(_NW, _NCH_SC, _CB)

    b1r = b1.reshape(1, 128)
    p, tab = _prep(geom, aux, W1, b1r)

    eg4 = _sc_gather_call(tab, src)
    eg = eg4.reshape(_K, _NP, 256)

    st1 = _stage1(eg, p)
    h2, st2 = _stage2(eg, p, st1, W2, b2.reshape(1, 128),
                      bn1_g.reshape(1, 128), bn1_b.reshape(1, 128))
    out = _stage3(h2, eg, aux, st2, Wa1, ba1.reshape(1, 64), Wa2,
                  ba2.reshape(1, 256), bn2_g.reshape(1, 128),
                  bn2_b.reshape(1, 128), ln_g.reshape(1, 128),
                  ln_b.reshape(1, 128))
    return out
